# baseline (speedup 1.0000x reference)
import jax
import jax.numpy as jnp
from jax.experimental import pallas as pl
from jax.experimental.pallas import tpu as pltpu

_N = 9216
_D = 256
_K = 1024
_BN = 512
_NB = _N // _BN


def _vq_kernel(z_ref, c_ref, zq_ref, idx_ref, commit_ref, csq_ref, acc_ref):
    i = pl.program_id(0)
    zb = z_ref[...]
    cb = c_ref[...]

    @pl.when(i == 0)
    def _init():
        csq_ref[...] = jnp.sum(cb * cb, axis=1).reshape(1, _K)
        acc_ref[...] = jnp.zeros((1, 1), jnp.float32)

    z_sq = jnp.sum(zb * zb, axis=1, keepdims=True)
    m = jnp.dot(zb, cb.T, preferred_element_type=jnp.float32)
    dist = z_sq - 2.0 * m + csq_ref[...]
    idx = jnp.argmin(dist, axis=1).astype(jnp.int32)
    oh = (idx[:, None] == jax.lax.broadcasted_iota(jnp.int32, (_BN, _K), 1))
    zq = jnp.dot(oh.astype(jnp.float32), cb,
                 preferred_element_type=jnp.float32)
    zq_ref[...] = zb + (zq - zb)
    idx_ref[...] = idx.reshape(1, 1, _BN)
    diff = zb - zq
    part = jnp.sum(diff * diff).reshape(1, 1)
    acc_ref[...] += part

    @pl.when(i == _NB - 1)
    def _fin():
        commit_ref[...] = acc_ref[...] * (1.0 / (_N * _D))


def kernel(z, codebook):
    zq, idx3, commit = pl.pallas_call(
        _vq_kernel,
        grid=(_NB,),
        in_specs=[
            pl.BlockSpec((_BN, _D), lambda i: (i, 0)),
            pl.BlockSpec((_K, _D), lambda i: (0, 0)),
        ],
        out_specs=[
            pl.BlockSpec((_BN, _D), lambda i: (i, 0)),
            pl.BlockSpec((1, 1, _BN), lambda i: (i, 0, 0)),
            pl.BlockSpec((1, 1), lambda i: (0, 0)),
        ],
        out_shape=[
            jax.ShapeDtypeStruct((_N, _D), jnp.float32),
            jax.ShapeDtypeStruct((_NB, 1, _BN), jnp.int32),
            jax.ShapeDtypeStruct((1, 1), jnp.float32),
        ],
        scratch_shapes=[
            pltpu.VMEM((1, _K), jnp.float32),
            pltpu.VMEM((1, 1), jnp.float32),
        ],
    )(z, codebook)
    return (zq, idx3.reshape(_N), commit.reshape(()))
